# baseline (speedup 1.0000x reference)
import functools

import jax
import jax.numpy as jnp
from jax import lax
from jax.experimental import pallas as pl
from jax.experimental.pallas import tpu as pltpu
from jax.experimental.pallas import tpu_sc as plsc

B = 16384
F = 26
V = 100000
D = 16

N = B * F
NC = 2
NS = 16
NW = NC * NS
RPW = N // NW
CH = 128
CPW = RPW // CH
GRP = 8
NGRP = CPW // GRP


def _body(idx_hbm, tab_hbm, out_hbm, idx_v, rows_v, gsem):
    cid = lax.axis_index("c")
    sid = lax.axis_index("s")
    wid = sid * NC + cid
    chunk0 = wid * CPW

    pltpu.sync_copy(idx_hbm.at[pl.ds(chunk0, CPW)], idx_v)

    lane = lax.iota(jnp.int32, 16)

    @pl.loop(0, CPW)
    def _offsets(c):
        pbase = (chunk0 + c) * CH
        for r in range(CH // 16):
            p = pbase + r * 16 + lane
            f = lax.rem(p, F)
            idx_v[c, pl.ds(r * 16, 16)] = idx_v[c, pl.ds(r * 16, 16)] + f * V

    @pl.loop(0, NGRP)
    def _groups(g):
        descs = []
        for j in range(GRP):
            c = g * GRP + j
            descs.append(
                pltpu.async_copy(
                    tab_hbm.at[idx_v.at[c]], rows_v.at[pl.ds(j * CH, CH)], gsem
                )
            )
        for d in descs:
            d.wait()
        row0 = (chunk0 + g * GRP) * CH
        pltpu.sync_copy(rows_v, out_hbm.at[pl.ds(row0, GRP * CH)])


_lookup = functools.partial(
    pl.kernel,
    out_type=jax.ShapeDtypeStruct((N, D), jnp.float32),
    mesh=plsc.VectorSubcoreMesh(
        core_axis_name="c", subcore_axis_name="s", num_cores=NC, num_subcores=NS
    ),
    scratch_types=[
        pltpu.VMEM((CPW, CH), jnp.int32),
        pltpu.VMEM((GRP * CH, D), jnp.float32),
        pltpu.SemaphoreType.DMA,
    ],
    compiler_params=pltpu.CompilerParams(use_tc_tiling_on_sc=False),
)(_body)


@jax.jit
def kernel(indices, tables):
    idx = indices.reshape(N // CH, CH)
    tab = tables.reshape(F * V, D)
    out = _lookup(idx, tab)
    return out.reshape(B, F, D)


# speedup vs baseline: 7.1818x; 7.1818x over previous
import functools

import jax
import jax.numpy as jnp
from jax import lax
from jax.experimental import pallas as pl
from jax.experimental.pallas import tpu as pltpu
from jax.experimental.pallas import tpu_sc as plsc

B = 16384
F = 26
V = 100000
D = 16

NC = 2
NS = 16
NW = NC * NS
P = F * D
PPW = P // NW
CH = 2048


def _body(idx_hbm, tab_hbm, out_hbm, idx_v, plane_v, obuf_v):
    cid = lax.axis_index("c")
    sid = lax.axis_index("s")
    w = sid * NC + cid
    p0 = w * PPW

    @pl.loop(0, PPW)
    def _planes(i):
        p = p0 + i
        f = p // D
        d = p - f * D

        @pl.when(jnp.logical_or(i == 0, d == 0))
        def _load_idx():
            pltpu.sync_copy(idx_hbm.at[f], idx_v)

        pltpu.sync_copy(tab_hbm.at[f, d], plane_v)

        @pl.loop(0, B // CH)
        def _chunks(c):
            for j in range(CH // 16):
                ids = idx_v[pl.ds(c * CH + j * 16, 16)]
                obuf_v[pl.ds(j * 16, 16)] = plsc.load_gather(plane_v, [ids])
            pltpu.sync_copy(obuf_v, out_hbm.at[f, d, pl.ds(c * CH, CH)])


_lookup = functools.partial(
    pl.kernel,
    out_type=jax.ShapeDtypeStruct((F, D, B), jnp.float32),
    mesh=plsc.VectorSubcoreMesh(
        core_axis_name="c", subcore_axis_name="s", num_cores=NC, num_subcores=NS
    ),
    scratch_types=[
        pltpu.VMEM((B,), jnp.int32),
        pltpu.VMEM((V,), jnp.float32),
        pltpu.VMEM((CH,), jnp.float32),
    ],
    compiler_params=pltpu.CompilerParams(
        use_tc_tiling_on_sc=True, needs_layout_passes=False
    ),
)(_body)


@jax.jit
def kernel(indices, tables):
    idx_t = indices.T
    tab_t = tables.transpose(0, 2, 1)
    out_t = _lookup(idx_t, tab_t)
    return out_t.transpose(2, 0, 1)
